# Initial kernel scaffold; baseline (speedup 1.0000x reference)
import jax
import jax.numpy as jnp
from jax.experimental import pallas as pl


def kernel(x, x_sub, pos, pos_sub, w_sub, g_sub, be_sub, w, g, be):
    raise NotImplementedError("write your pallas kernel here")


# speedup vs baseline: 1.3031x; 1.3031x over previous
import functools

import jax
import jax.numpy as jnp
from jax.experimental import pallas as pl
from jax.experimental.pallas import tpu as pltpu

_EPS_BN = 1e-5
_PAD_POS = 1e6
_MASKED = 1e30


def _round_up(n, m):
    return ((n + m - 1) // m) * m


def _sub_branch_kernel(xs_ref, w_ref, g_ref, b_ref, h_ref, *, s_true):
    h = jax.lax.dot_general(xs_ref[...], w_ref[...], (((1,), (0,)), ((), ())),
                            preferred_element_type=jnp.float32)
    inv = 1.0 / s_true
    mean = jnp.sum(h, axis=0, keepdims=True) * inv
    ex2 = jnp.sum(h * h, axis=0, keepdims=True) * inv
    var = jnp.maximum(ex2 - mean * mean, 0.0)
    scale = g_ref[...] * jax.lax.rsqrt(var + _EPS_BN)
    shift = b_ref[...] - mean * scale
    h_ref[...] = jnp.maximum(h * scale + shift, 0.0).astype(jnp.bfloat16)


def _x_stats_kernel(x_ref, w_ref, g_ref, b_ref, ss_ref, acc_ref, *,
                    n_true, n_steps):
    @pl.when(pl.program_id(0) == 0)
    def _init():
        acc_ref[...] = jnp.zeros_like(acc_ref)

    h = jax.lax.dot_general(x_ref[...], w_ref[...], (((1,), (0,)), ((), ())),
                            preferred_element_type=jnp.float32)
    acc_ref[0:1, :] += jnp.sum(h, axis=0, keepdims=True)
    acc_ref[1:2, :] += jnp.sum(h * h, axis=0, keepdims=True)

    @pl.when(pl.program_id(0) == n_steps - 1)
    def _fold():
        inv = 1.0 / n_true
        mean = acc_ref[0:1, :] * inv
        var = jnp.maximum(acc_ref[1:2, :] * inv - mean * mean, 0.0)
        scale = g_ref[...] * jax.lax.rsqrt(var + _EPS_BN)
        ss_ref[0:1, :] = scale
        ss_ref[1:2, :] = b_ref[...] - mean * scale


def _fused_kernel(ps_ref, pq_ref, hs_ref, x_ref, w_ref, ss_ref, o_ref):
    d2 = None
    for d in range(3):
        diff = ps_ref[:, d:d + 1] - pq_ref[d:d + 1, :]
        d2 = diff * diff if d2 is None else d2 + diff * diff

    m = jnp.min(d2, axis=0, keepdims=True)
    t = jnp.where(d2 == m, _MASKED, d2)
    m = jnp.min(t, axis=0, keepdims=True)
    t = jnp.where(t == m, _MASKED, t)
    m3 = jnp.min(t, axis=0, keepdims=True)

    wts = jnp.where(d2 <= m3,
                    pl.reciprocal(jnp.maximum(d2, 1e-16), approx=True),
                    0.0)
    den = jnp.sum(wts, axis=0, keepdims=True)
    wn = (wts * pl.reciprocal(den, approx=True)).astype(jnp.bfloat16)

    interp = jax.lax.dot_general(wn, hs_ref[...], (((0,), (0,)), ((), ())),
                                 preferred_element_type=jnp.float32)

    h = jax.lax.dot_general(x_ref[...], w_ref[...], (((1,), (0,)), ((), ())),
                            preferred_element_type=jnp.float32)
    h = jnp.maximum(h * ss_ref[0:1, :] + ss_ref[1:2, :], 0.0)
    o_ref[...] = h + interp


def kernel(x, x_sub, pos, pos_sub, w_sub, g_sub, be_sub, w, g, be):
    f32 = jnp.float32
    n, c = x.shape
    s, _ = x_sub.shape

    tile = 128
    stats_tile = 8192
    s_pad = _round_up(max(s, 8), 128)
    n_pad = _round_up(n, stats_tile)

    x_p = jnp.pad(x.astype(f32), ((0, n_pad - n), (0, 0)))
    pos_qt = jnp.pad(pos.astype(f32).T, ((0, 0), (0, n_pad - n)))
    xs_p = jnp.pad(x_sub.astype(f32), ((0, s_pad - s), (0, 0)))
    ps_p = jnp.pad(pos_sub.astype(f32), ((0, s_pad - s), (0, 0)),
                   constant_values=_PAD_POS)

    w_sub_f = w_sub.astype(f32)
    w_f = w.astype(f32)
    g_sub_r = g_sub.astype(f32).reshape(1, c)
    b_sub_r = be_sub.astype(f32).reshape(1, c)
    g_r = g.astype(f32).reshape(1, c)
    b_r = be.astype(f32).reshape(1, c)

    c_in = w_sub_f.shape[0]

    h_sub = pl.pallas_call(
        functools.partial(_sub_branch_kernel, s_true=s),
        out_shape=jax.ShapeDtypeStruct((s_pad, c), jnp.bfloat16),
        grid=(1,),
        in_specs=[pl.BlockSpec((s_pad, c_in), lambda i: (0, 0)),
                  pl.BlockSpec((c_in, c), lambda i: (0, 0)),
                  pl.BlockSpec((1, c), lambda i: (0, 0)),
                  pl.BlockSpec((1, c), lambda i: (0, 0))],
        out_specs=pl.BlockSpec((s_pad, c), lambda i: (0, 0)),
    )(xs_p, w_sub_f, g_sub_r, b_sub_r)

    n_steps = n_pad // stats_tile
    ss_x = pl.pallas_call(
        functools.partial(_x_stats_kernel, n_true=n, n_steps=n_steps),
        out_shape=jax.ShapeDtypeStruct((2, c), f32),
        grid=(n_steps,),
        in_specs=[pl.BlockSpec((stats_tile, c), lambda i: (i, 0)),
                  pl.BlockSpec((c, c), lambda i: (0, 0)),
                  pl.BlockSpec((1, c), lambda i: (0, 0)),
                  pl.BlockSpec((1, c), lambda i: (0, 0))],
        out_specs=pl.BlockSpec((2, c), lambda i: (0, 0)),
        scratch_shapes=[pltpu.VMEM((2, c), f32)],
        compiler_params=pltpu.CompilerParams(
            dimension_semantics=("arbitrary",)),
    )(x_p, w_f, g_r, b_r)

    out = pl.pallas_call(
        _fused_kernel,
        out_shape=jax.ShapeDtypeStruct((n_pad, c), f32),
        grid=(n_pad // tile,),
        in_specs=[
            pl.BlockSpec((s_pad, 3), lambda i: (0, 0)),
            pl.BlockSpec((3, tile), lambda i: (0, i)),
            pl.BlockSpec((s_pad, c), lambda i: (0, 0)),
            pl.BlockSpec((tile, c), lambda i: (i, 0)),
            pl.BlockSpec((c, c), lambda i: (0, 0)),
            pl.BlockSpec((2, c), lambda i: (0, 0)),
        ],
        out_specs=pl.BlockSpec((tile, c), lambda i: (i, 0)),
        compiler_params=pltpu.CompilerParams(
            dimension_semantics=("parallel",),
            vmem_limit_bytes=48 * 1024 * 1024),
    )(ps_p, pos_qt, h_sub, x_p, w_f, ss_x)

    return out[:n]


# speedup vs baseline: 2.0442x; 1.5687x over previous
import functools

import jax
import jax.numpy as jnp
from jax.experimental import pallas as pl
from jax.experimental.pallas import tpu as pltpu

_EPS_BN = 1e-5
_PAD_POS = 1e6
_MASKED = 1e30


def _round_up(n, m):
    return ((n + m - 1) // m) * m


def _sub_branch_kernel(xs_ref, w_ref, g_ref, b_ref, h_ref, *, s_true):
    h = jax.lax.dot_general(xs_ref[...], w_ref[...], (((1,), (0,)), ((), ())),
                            preferred_element_type=jnp.float32)
    c = h.shape[1]
    inv = 1.0 / s_true
    mean = jnp.sum(h, axis=0, keepdims=True) * inv
    ex2 = jnp.sum(h * h, axis=0, keepdims=True) * inv
    var = jnp.maximum(ex2 - mean * mean, 0.0)
    scale = g_ref[...] * jax.lax.rsqrt(var + _EPS_BN)
    shift = b_ref[...] - mean * scale
    h_ref[:, :c] = jnp.maximum(h * scale + shift, 0.0).astype(jnp.bfloat16)
    h_ref[:, c:c + 1] = jnp.ones((h.shape[0], 1), jnp.bfloat16)


def _x_stats_kernel(x_ref, w_ref, g_ref, b_ref, ss_ref, acc_ref, *,
                    n_true, n_steps):
    @pl.when(pl.program_id(0) == 0)
    def _init():
        acc_ref[...] = jnp.zeros_like(acc_ref)

    h = jax.lax.dot_general(x_ref[...], w_ref[...], (((1,), (0,)), ((), ())),
                            preferred_element_type=jnp.float32)
    acc_ref[0:1, :] += jnp.sum(h, axis=0, keepdims=True)
    acc_ref[1:2, :] += jnp.sum(h * h, axis=0, keepdims=True)

    @pl.when(pl.program_id(0) == n_steps - 1)
    def _fold():
        inv = 1.0 / n_true
        mean = acc_ref[0:1, :] * inv
        var = jnp.maximum(acc_ref[1:2, :] * inv - mean * mean, 0.0)
        scale = g_ref[...] * jax.lax.rsqrt(var + _EPS_BN)
        ss_ref[0:1, :] = scale
        ss_ref[1:2, :] = b_ref[...] - mean * scale


def _fused_kernel(ps_ref, pq_ref, hs_ref, x_ref, w_ref, ss_ref, o_ref, d2_ref,
                  *, strip):
    s_pad = ps_ref.shape[0]
    qx = pq_ref[0:1, :]
    qy = pq_ref[1:2, :]
    qz = pq_ref[2:3, :]

    a = b = c3 = None
    for i in range(s_pad // strip):
        ps = ps_ref[i * strip:(i + 1) * strip, :]
        dx = ps[:, 0:1] - qx
        dy = ps[:, 1:2] - qy
        dz = ps[:, 2:3] - qz
        d2 = dx * dx + dy * dy + dz * dz
        d2_ref[i * strip:(i + 1) * strip, :] = d2
        if a is None:
            a = d2
            b = jnp.full_like(d2, _MASKED)
            c3 = jnp.full_like(d2, _MASKED)
        else:
            hi1 = jnp.maximum(a, d2)
            a = jnp.minimum(a, d2)
            hi2 = jnp.maximum(b, hi1)
            b = jnp.minimum(b, hi1)
            c3 = jnp.minimum(c3, hi2)

    cand = jnp.concatenate([a, b, c3], axis=0)
    m = jnp.min(cand, axis=0, keepdims=True)
    t = jnp.where(cand <= m, _MASKED, cand)
    m = jnp.min(t, axis=0, keepdims=True)
    t = jnp.where(t <= m, _MASKED, t)
    m3 = jnp.min(t, axis=0, keepdims=True)

    d2f = d2_ref[...]
    wn = jnp.where(d2f <= m3,
                   pl.reciprocal(jnp.maximum(d2f, 1e-16), approx=True),
                   0.0).astype(jnp.bfloat16)

    num = jax.lax.dot_general(wn, hs_ref[...], (((0,), (0,)), ((), ())),
                              preferred_element_type=jnp.float32)
    cc = num.shape[1] - 1
    interp = num[:, :cc] * pl.reciprocal(num[:, cc:cc + 1], approx=True)

    h = jax.lax.dot_general(x_ref[...], w_ref[...], (((1,), (0,)), ((), ())),
                            preferred_element_type=jnp.float32)
    h = jnp.maximum(h * ss_ref[0:1, :] + ss_ref[1:2, :], 0.0)
    o_ref[...] = h + interp


def kernel(x, x_sub, pos, pos_sub, w_sub, g_sub, be_sub, w, g, be):
    f32 = jnp.float32
    n, c = x.shape
    s, _ = x_sub.shape

    tile = 128
    stats_tile = 8192
    s_pad = _round_up(max(s, 8), 128)
    n_pad = _round_up(n, stats_tile)

    x_p = jnp.pad(x.astype(f32), ((0, n_pad - n), (0, 0)))
    pos_qt = jnp.pad(pos.astype(f32).T, ((0, 0), (0, n_pad - n)))
    xs_p = jnp.pad(x_sub.astype(f32), ((0, s_pad - s), (0, 0)))
    ps_p = jnp.pad(pos_sub.astype(f32), ((0, s_pad - s), (0, 0)),
                   constant_values=_PAD_POS)

    w_sub_f = w_sub.astype(f32)
    w_f = w.astype(f32)
    g_sub_r = g_sub.astype(f32).reshape(1, c)
    b_sub_r = be_sub.astype(f32).reshape(1, c)
    g_r = g.astype(f32).reshape(1, c)
    b_r = be.astype(f32).reshape(1, c)

    c_in = w_sub_f.shape[0]

    h_sub = pl.pallas_call(
        functools.partial(_sub_branch_kernel, s_true=s),
        out_shape=jax.ShapeDtypeStruct((s_pad, c + 1), jnp.bfloat16),
        grid=(1,),
        in_specs=[pl.BlockSpec((s_pad, c_in), lambda i: (0, 0)),
                  pl.BlockSpec((c_in, c), lambda i: (0, 0)),
                  pl.BlockSpec((1, c), lambda i: (0, 0)),
                  pl.BlockSpec((1, c), lambda i: (0, 0))],
        out_specs=pl.BlockSpec((s_pad, c + 1), lambda i: (0, 0)),
    )(xs_p, w_sub_f, g_sub_r, b_sub_r)

    n_steps = n_pad // stats_tile
    ss_x = pl.pallas_call(
        functools.partial(_x_stats_kernel, n_true=n, n_steps=n_steps),
        out_shape=jax.ShapeDtypeStruct((2, c), f32),
        grid=(n_steps,),
        in_specs=[pl.BlockSpec((stats_tile, c), lambda i: (i, 0)),
                  pl.BlockSpec((c, c), lambda i: (0, 0)),
                  pl.BlockSpec((1, c), lambda i: (0, 0)),
                  pl.BlockSpec((1, c), lambda i: (0, 0))],
        out_specs=pl.BlockSpec((2, c), lambda i: (0, 0)),
        scratch_shapes=[pltpu.VMEM((2, c), f32)],
        compiler_params=pltpu.CompilerParams(
            dimension_semantics=("arbitrary",)),
    )(x_p, w_f, g_r, b_r)

    out = pl.pallas_call(
        functools.partial(_fused_kernel, strip=128),
        out_shape=jax.ShapeDtypeStruct((n_pad, c), f32),
        grid=(n_pad // tile,),
        in_specs=[
            pl.BlockSpec((s_pad, 3), lambda i: (0, 0)),
            pl.BlockSpec((3, tile), lambda i: (0, i)),
            pl.BlockSpec((s_pad, c + 1), lambda i: (0, 0)),
            pl.BlockSpec((tile, c), lambda i: (i, 0)),
            pl.BlockSpec((c, c), lambda i: (0, 0)),
            pl.BlockSpec((2, c), lambda i: (0, 0)),
        ],
        out_specs=pl.BlockSpec((tile, c), lambda i: (i, 0)),
        scratch_shapes=[pltpu.VMEM((s_pad, tile), f32)],
        compiler_params=pltpu.CompilerParams(
            dimension_semantics=("parallel",),
            vmem_limit_bytes=48 * 1024 * 1024),
    )(ps_p, pos_qt, h_sub, x_p, w_f, ss_x)

    return out[:n]
